# Initial kernel scaffold; baseline (speedup 1.0000x reference)
import jax
import jax.numpy as jnp
from jax.experimental import pallas as pl


def kernel(x, edge_index, edge_attr, params):
    raise NotImplementedError("write your pallas kernel here")


# speedup vs baseline: 1.8566x; 1.8566x over previous
import functools

import jax
import jax.numpy as jnp
from jax import lax
from jax.experimental import pallas as pl
from jax.experimental.pallas import tpu as pltpu
from jax.experimental.pallas import tpu_sc as plsc

_N = 10000
_E = 160000
_H = 256
_HALF = 128
_NC = 2
_NS = 16
_C = 80
_EPT = _E // _NS
_LANES = 16
_F32 = jnp.float32


def _sc_mesh():
    return plsc.VectorSubcoreMesh(
        core_axis_name="c", subcore_axis_name="s",
        num_cores=_NC, num_subcores=_NS)


def _sc_aggr_body(h_lo, h_hi, src_h, dst_h, ep_lo_h, ep_hi_h,
                  zeros_h, out_lo, out_hi,
                  sidx_v, didx_v, rows_v, ep_v, acc_sh, sem, sem_e):
    ci = lax.axis_index("c")
    si = lax.axis_index("s")
    base = si * _EPT

    @pl.when(si == 0)
    def _():
        pltpu.sync_copy(zeros_h, acc_sh)
    plsc.subcore_barrier()

    def run(h_half, ep_half):
        def chunk_body(k, carry):
            off = base + k * _C
            pltpu.sync_copy(src_h.at[pl.ds(off, _C)], sidx_v)
            pltpu.sync_copy(dst_h.at[pl.ds(off, _C)], didx_v)
            cp_e = pltpu.async_copy(ep_half.at[pl.ds(off, _C)], ep_v, sem_e)
            cp_h = pltpu.async_copy(h_half.at[sidx_v], rows_v, sem)
            cp_e.wait()
            cp_h.wait()

            def edge_body(i, c2):
                for j in range(_HALF // _LANES):
                    sl = pl.ds(j * _LANES, _LANES)
                    t = jnp.maximum(rows_v[i, sl] + ep_v[i, sl], 0.0)
                    rows_v[i, sl] = t
                return c2

            lax.fori_loop(0, _C, edge_body, 0)
            pltpu.sync_copy(rows_v, acc_sh.at[didx_v], add=True)
            return carry

        lax.fori_loop(0, _EPT // _C, chunk_body, 0)

    @pl.when(ci == 0)
    def _():
        run(h_lo, ep_lo_h)

    @pl.when(ci == 1)
    def _():
        run(h_hi, ep_hi_h)

    plsc.subcore_barrier()

    @pl.when(jnp.logical_and(si == 0, ci == 0))
    def _():
        pltpu.sync_copy(acc_sh, out_lo)

    @pl.when(jnp.logical_and(si == 0, ci == 1))
    def _():
        pltpu.sync_copy(acc_sh, out_hi)


def _sc_aggr(h_lo, h_hi, src, dst, ep_lo, ep_hi, zeros):
    kern = pl.kernel(
        _sc_aggr_body,
        out_type=[jax.ShapeDtypeStruct((_N, _HALF), _F32),
                  jax.ShapeDtypeStruct((_N, _HALF), _F32)],
        mesh=_sc_mesh(),
        scratch_types=[
            pltpu.VMEM((_C,), jnp.int32),
            pltpu.VMEM((_C,), jnp.int32),
            pltpu.VMEM((_C, _HALF), _F32),
            pltpu.VMEM((_C, _HALF), _F32),
            pltpu.VMEM_SHARED((_N, _HALF), _F32),
            pltpu.SemaphoreType.DMA,
            pltpu.SemaphoreType.DMA,
        ],
    )
    return kern(h_lo, h_hi, src, dst, ep_lo, ep_hi, zeros)


def _sc_head_body(p_lo, p_hi, q_lo, q_hi, src_h, dst_h, a0_h, a1_h,
                  uvc_lo_h, uvc_hi_h, g_lo, g_hi,
                  sidx_v, didx_v, a0_v, a1_v, prow_v, qrow_v, uvc_v,
                  sem_p, sem_q):
    ci = lax.axis_index("c")
    si = lax.axis_index("s")
    base = si * _EPT

    def run(p_half, q_half, uvc_h, g_half):
        pltpu.sync_copy(uvc_h, uvc_v)
        nj = _HALF // _LANES
        us = [uvc_v[0, pl.ds(j * _LANES, _LANES)] for j in range(nj)]
        vs = [uvc_v[1, pl.ds(j * _LANES, _LANES)] for j in range(nj)]
        cs = [uvc_v[2, pl.ds(j * _LANES, _LANES)] for j in range(nj)]

        def chunk_body(k, carry):
            off = base + k * _C
            pltpu.sync_copy(src_h.at[pl.ds(off, _C)], sidx_v)
            pltpu.sync_copy(dst_h.at[pl.ds(off, _C)], didx_v)
            pltpu.sync_copy(a0_h.at[pl.ds(off, _C)], a0_v)
            pltpu.sync_copy(a1_h.at[pl.ds(off, _C)], a1_v)
            cp_p = pltpu.async_copy(p_half.at[sidx_v], prow_v, sem_p)
            cp_q = pltpu.async_copy(q_half.at[didx_v], qrow_v, sem_q)
            cp_p.wait()
            cp_q.wait()

            def group_body(g, c2):
                gbase = g * _LANES
                a0g = a0_v[pl.ds(gbase, _LANES)]
                a1g = a1_v[pl.ds(gbase, _LANES)]
                for l in range(_LANES):
                    i = gbase + l
                    a0s = a0g[l]
                    a1s = a1g[l]
                    for j in range(nj):
                        sl = pl.ds(j * _LANES, _LANES)
                        t = prow_v[i, sl] + qrow_v[i, sl]
                        t = jnp.maximum(
                            t + a0s * us[j] + (a1s * vs[j] + cs[j]), 0.0)
                        prow_v[i, sl] = t
                return c2

            lax.fori_loop(0, _C // _LANES, group_body, 0)
            pltpu.sync_copy(prow_v, g_half.at[pl.ds(off, _C)])
            return carry

        lax.fori_loop(0, _EPT // _C, chunk_body, 0)

    @pl.when(ci == 0)
    def _():
        run(p_lo, q_lo, uvc_lo_h, g_lo)

    @pl.when(ci == 1)
    def _():
        run(p_hi, q_hi, uvc_hi_h, g_hi)


def _sc_head_edges(p_lo, p_hi, q_lo, q_hi, src, dst, a0, a1, uvc_lo, uvc_hi):
    kern = pl.kernel(
        _sc_head_body,
        out_type=[jax.ShapeDtypeStruct((_E, _HALF), _F32),
                  jax.ShapeDtypeStruct((_E, _HALF), _F32)],
        mesh=_sc_mesh(),
        scratch_types=[
            pltpu.VMEM((_C,), jnp.int32),
            pltpu.VMEM((_C,), jnp.int32),
            pltpu.VMEM((_C,), _F32),
            pltpu.VMEM((_C,), _F32),
            pltpu.VMEM((_C, _HALF), _F32),
            pltpu.VMEM((_C, _HALF), _F32),
            pltpu.VMEM((3, _HALF), _F32),
            pltpu.SemaphoreType.DMA,
            pltpu.SemaphoreType.DMA,
        ],
    )
    return kern(p_lo, p_hi, q_lo, q_hi, src, dst, a0, a1, uvc_lo, uvc_hi)


_BN = 1000
_BE = 2000


def _init_body(x_ref, w_ref, b_ref, lo_ref, hi_ref):
    acc = jnp.dot(x_ref[...], w_ref[...], preferred_element_type=_F32)
    acc = acc + b_ref[...]
    lo_ref[...] = acc[:, :_HALF]
    hi_ref[...] = acc[:, _HALF:]


def _tc_init(x, w, b):
    return pl.pallas_call(
        _init_body,
        grid=(_N // _BN,),
        in_specs=[
            pl.BlockSpec((_BN, _H), lambda i: (i, 0)),
            pl.BlockSpec((_H, _H), lambda i: (0, 0)),
            pl.BlockSpec((1, _H), lambda i: (0, 0)),
        ],
        out_specs=[
            pl.BlockSpec((_BN, _HALF), lambda i: (i, 0)),
            pl.BlockSpec((_BN, _HALF), lambda i: (i, 0)),
        ],
        out_shape=[jax.ShapeDtypeStruct((_N, _HALF), _F32),
                   jax.ShapeDtypeStruct((_N, _HALF), _F32)],
    )(x, w, b)


def _eproj_body(ea_ref, ew_ref, eb_ref, lw0_ref, lb0_ref, lw1_ref, lb1_ref,
                lw2_ref, lb2_ref, *out_refs):
    e = jnp.dot(ea_ref[...], ew_ref[...],
                preferred_element_type=_F32) + eb_ref[...]
    for l, (lw, lb) in enumerate(((lw0_ref, lb0_ref), (lw1_ref, lb1_ref),
                                  (lw2_ref, lb2_ref))):
        ep = jnp.dot(e, lw[...], preferred_element_type=_F32) + lb[...]
        out_refs[2 * l][...] = ep[:, :_HALF]
        out_refs[2 * l + 1][...] = ep[:, _HALF:]


def _tc_eproj(ea, ew, eb, lws, lbs):
    return pl.pallas_call(
        _eproj_body,
        grid=(_E // _BE,),
        in_specs=[
            pl.BlockSpec((_BE, 2), lambda i: (i, 0)),
            pl.BlockSpec((2, _H), lambda i: (0, 0)),
            pl.BlockSpec((1, _H), lambda i: (0, 0)),
            pl.BlockSpec((_H, _H), lambda i: (0, 0)),
            pl.BlockSpec((1, _H), lambda i: (0, 0)),
            pl.BlockSpec((_H, _H), lambda i: (0, 0)),
            pl.BlockSpec((1, _H), lambda i: (0, 0)),
            pl.BlockSpec((_H, _H), lambda i: (0, 0)),
            pl.BlockSpec((1, _H), lambda i: (0, 0)),
        ],
        out_specs=[pl.BlockSpec((_BE, _HALF), lambda i: (i, 0))] * 6,
        out_shape=[jax.ShapeDtypeStruct((_E, _HALF), _F32)] * 6,
    )(ea, ew, eb, lws[0], lbs[0], lws[1], lbs[1], lws[2], lbs[2])


def _mlp_body(hlo_ref, hhi_ref, alo_ref, ahi_ref, m1_ref, b1_ref,
              m2_ref, b2_ref, z2_ref, stats_ref):
    z = jnp.concatenate(
        [hlo_ref[...] + alo_ref[...], hhi_ref[...] + ahi_ref[...]], axis=1)
    t = jnp.dot(z, m1_ref[...], preferred_element_type=_F32) + b1_ref[...]
    t = jnp.maximum(t, 0.0)
    zz = jnp.dot(t, m2_ref[...], preferred_element_type=_F32) + b2_ref[...]
    z2_ref[...] = zz
    s = jnp.sum(zz, axis=0, keepdims=True)
    sq = jnp.sum(zz * zz, axis=0, keepdims=True)
    st = jnp.concatenate([s, sq], axis=0)

    @pl.when(pl.program_id(0) == 0)
    def _():
        stats_ref[...] = st

    @pl.when(pl.program_id(0) != 0)
    def _():
        stats_ref[...] = stats_ref[...] + st


def _tc_mlp(h_lo, h_hi, a_lo, a_hi, m1, b1, m2, b2):
    return pl.pallas_call(
        _mlp_body,
        grid=(_N // _BN,),
        in_specs=[
            pl.BlockSpec((_BN, _HALF), lambda i: (i, 0)),
            pl.BlockSpec((_BN, _HALF), lambda i: (i, 0)),
            pl.BlockSpec((_BN, _HALF), lambda i: (i, 0)),
            pl.BlockSpec((_BN, _HALF), lambda i: (i, 0)),
            pl.BlockSpec((_H, _H), lambda i: (0, 0)),
            pl.BlockSpec((1, _H), lambda i: (0, 0)),
            pl.BlockSpec((_H, _H), lambda i: (0, 0)),
            pl.BlockSpec((1, _H), lambda i: (0, 0)),
        ],
        out_specs=[
            pl.BlockSpec((_BN, _H), lambda i: (i, 0)),
            pl.BlockSpec((2, _H), lambda i: (0, 0)),
        ],
        out_shape=[jax.ShapeDtypeStruct((_N, _H), _F32),
                   jax.ShapeDtypeStruct((2, _H), _F32)],
    )(h_lo, h_hi, a_lo, a_hi, m1, b1, m2, b2)


def _bn_body(z2_ref, hlo_ref, hhi_ref, mu_ref, sc_ref, be_ref,
             lo_ref, hi_ref):
    h = jnp.concatenate([hlo_ref[...], hhi_ref[...]], axis=1)
    zn = (z2_ref[...] - mu_ref[...]) * sc_ref[...] + be_ref[...]
    nh = jnp.maximum(zn + h, 0.0)
    lo_ref[...] = nh[:, :_HALF]
    hi_ref[...] = nh[:, _HALF:]


def _tc_bn_residual(z2, h_lo, h_hi, mu, scale, beta):
    return pl.pallas_call(
        _bn_body,
        grid=(_N // _BN,),
        in_specs=[
            pl.BlockSpec((_BN, _H), lambda i: (i, 0)),
            pl.BlockSpec((_BN, _HALF), lambda i: (i, 0)),
            pl.BlockSpec((_BN, _HALF), lambda i: (i, 0)),
            pl.BlockSpec((1, _H), lambda i: (0, 0)),
            pl.BlockSpec((1, _H), lambda i: (0, 0)),
            pl.BlockSpec((1, _H), lambda i: (0, 0)),
        ],
        out_specs=[
            pl.BlockSpec((_BN, _HALF), lambda i: (i, 0)),
            pl.BlockSpec((_BN, _HALF), lambda i: (i, 0)),
        ],
        out_shape=[jax.ShapeDtypeStruct((_N, _HALF), _F32),
                   jax.ShapeDtypeStruct((_N, _HALF), _F32)],
    )(z2, h_lo, h_hi, mu, scale, beta)


def _proj_body(hlo_ref, hhi_ref, wa_ref, ba_ref, wb_ref,
               plo_ref, phi_ref, qlo_ref, qhi_ref):
    h = jnp.concatenate([hlo_ref[...], hhi_ref[...]], axis=1)
    p = jnp.dot(h, wa_ref[...], preferred_element_type=_F32) + ba_ref[...]
    q = jnp.dot(h, wb_ref[...], preferred_element_type=_F32)
    plo_ref[...] = p[:, :_HALF]
    phi_ref[...] = p[:, _HALF:]
    qlo_ref[...] = q[:, :_HALF]
    qhi_ref[...] = q[:, _HALF:]


def _tc_head_proj(h_lo, h_hi, wa, ba, wb):
    return pl.pallas_call(
        _proj_body,
        grid=(_N // _BN,),
        in_specs=[
            pl.BlockSpec((_BN, _HALF), lambda i: (i, 0)),
            pl.BlockSpec((_BN, _HALF), lambda i: (i, 0)),
            pl.BlockSpec((_H, _H), lambda i: (0, 0)),
            pl.BlockSpec((1, _H), lambda i: (0, 0)),
            pl.BlockSpec((_H, _H), lambda i: (0, 0)),
        ],
        out_specs=[pl.BlockSpec((_BN, _HALF), lambda i: (i, 0))] * 4,
        out_shape=[jax.ShapeDtypeStruct((_N, _HALF), _F32)] * 4,
    )(h_lo, h_hi, wa, ba, wb)


def _tail_body(glo_ref, ghi_ref, w2_ref, b2_ref, w3_ref, b3_ref, out_ref):
    g = jnp.concatenate([glo_ref[...], ghi_ref[...]], axis=1)
    t = jnp.dot(g, w2_ref[...], preferred_element_type=_F32) + b2_ref[...]
    t = jnp.maximum(t, 0.0)
    out_ref[...] = jnp.dot(t, w3_ref[...], preferred_element_type=_F32) \
        + b3_ref[...]


def _tc_head_tail(g_lo, g_hi, w2, b2, w3, b3):
    hh = _H // 2
    out = 8
    return pl.pallas_call(
        _tail_body,
        grid=(_E // _BE,),
        in_specs=[
            pl.BlockSpec((_BE, _HALF), lambda i: (i, 0)),
            pl.BlockSpec((_BE, _HALF), lambda i: (i, 0)),
            pl.BlockSpec((_H, hh), lambda i: (0, 0)),
            pl.BlockSpec((1, hh), lambda i: (0, 0)),
            pl.BlockSpec((hh, out), lambda i: (0, 0)),
            pl.BlockSpec((1, out), lambda i: (0, 0)),
        ],
        out_specs=pl.BlockSpec((_BE, out), lambda i: (i, 0)),
        out_shape=jax.ShapeDtypeStruct((_E, out), _F32),
    )(g_lo, g_hi, w2, b2, w3, b3)


def kernel(x, edge_index, edge_attr, params):
    src = edge_index[0]
    dst = edge_index[1]
    a0 = edge_attr[:, 0]
    a1 = edge_attr[:, 1]
    zeros = jnp.zeros((_N, _HALF), _F32)

    h_lo, h_hi = _tc_init(x, params["node_W"], params["node_b"][None, :])

    eps = _tc_eproj(
        edge_attr, params["edge_W"], params["edge_b"][None, :],
        [lp["lin_W"] for lp in params["layers"]],
        [lp["lin_b"][None, :] for lp in params["layers"]])

    for li, lp in enumerate(params["layers"]):
        a_lo, a_hi = _sc_aggr(h_lo, h_hi, src, dst,
                              eps[2 * li], eps[2 * li + 1], zeros)

        z2, stats = _tc_mlp(h_lo, h_hi, a_lo, a_hi,
                            lp["m1_W"], lp["m1_b"][None, :],
                            lp["m2_W"], lp["m2_b"][None, :])
        mu = stats[0] / _N
        var = stats[1] / _N - mu * mu
        scale = lp["bn_g"] / jnp.sqrt(var + 1e-5)
        h_lo, h_hi = _tc_bn_residual(z2, h_lo, h_hi, mu[None, :],
                                     scale[None, :], lp["bn_b"][None, :])

    wa = params["h1_W"][:_H]
    wb = params["h1_W"][_H:2 * _H]
    uvc_head = jnp.concatenate(
        [params["h1_W"][2 * _H:], params["h1_b"][None, :]], axis=0)

    p_lo, p_hi, q_lo, q_hi = _tc_head_proj(
        h_lo, h_hi, wa, params["h1_b"][None, :] * 0.0, wb)

    g_lo, g_hi = _sc_head_edges(
        p_lo, p_hi, q_lo, q_hi, src, dst, a0, a1,
        uvc_head[:, :_HALF], uvc_head[:, _HALF:])

    return _tc_head_tail(g_lo, g_hi, params["h2_W"],
                         params["h2_b"][None, :], params["h3_W"],
                         params["h3_b"][None, :])


# speedup vs baseline: 3.2885x; 1.7713x over previous
import functools

import jax
import jax.numpy as jnp
from jax import lax
from jax.experimental import pallas as pl
from jax.experimental.pallas import tpu as pltpu
from jax.experimental.pallas import tpu_sc as plsc

_N = 10000
_E = 160000
_H = 256
_HALF = 128
_NC = 2
_NS = 16
_C = 40
_EPT = _E // _NS
_LANES = 16
_F32 = jnp.float32


def _sc_mesh():
    return plsc.VectorSubcoreMesh(
        core_axis_name="c", subcore_axis_name="s",
        num_cores=_NC, num_subcores=_NS)


_M = 5
_SUP = _C * _M
_NSUP = _EPT // _SUP
_CPS = _EPT // _C


_MA = 2
_SUPA = _C * _MA
_NSUPA = _EPT // _SUPA


def _sc_aggr_body(h_lo, h_hi, src_h, dst_h, ep_lo_h, ep_hi_h,
                  zeros_h, out_lo, out_hi, *sc):
    rows = sc[0:2]
    ep = sc[2:4]
    nidx = 4 * _MA
    sidx = tuple(sc[4 + s * _MA:4 + (s + 1) * _MA] for s in range(4))
    didx = tuple(sc[4 + nidx + s * _MA:4 + nidx + (s + 1) * _MA]
                 for s in range(4))
    acc_sh = sc[4 + 2 * nidx]
    semg = sc[5 + 2 * nidx:7 + 2 * nidx]
    seme = sc[7 + 2 * nidx:9 + 2 * nidx]
    sems = sc[9 + 2 * nidx:11 + 2 * nidx]
    semi = sc[11 + 2 * nidx:15 + 2 * nidx]

    ci = lax.axis_index("c")
    si = lax.axis_index("s")
    base = si * _EPT

    @pl.when(si == 0)
    def _():
        pltpu.sync_copy(zeros_h, acc_sh)
    plsc.subcore_barrier()

    def run(h_half, ep_half):
        def prefetch_idx(kk, b, s):
            off = base + kk * _SUPA
            for m in range(_MA):
                pltpu.async_copy(src_h.at[pl.ds(off + m * _C, _C)],
                                 sidx[s][m], semi[s])
                pltpu.async_copy(dst_h.at[pl.ds(off + m * _C, _C)],
                                 didx[s][m], semi[s])

        def issue(kk, b, s):
            off = base + kk * _SUPA
            pltpu.async_copy(ep_half.at[pl.ds(off, _SUPA)], ep[b], seme[b])
            for m in range(_MA):
                pltpu.make_async_copy(src_h.at[pl.ds(off + m * _C, _C)],
                                      sidx[s][m], semi[s]).wait()
                pltpu.make_async_copy(dst_h.at[pl.ds(off + m * _C, _C)],
                                      didx[s][m], semi[s]).wait()
                pltpu.async_copy(h_half.at[sidx[s][m]],
                                 rows[b].at[pl.ds(m * _C, _C)], semg[b])

        def finish(kk, b, s):
            off = base + kk * _SUPA
            pltpu.make_async_copy(
                ep_half.at[pl.ds(off, _SUPA)], ep[b], seme[b]).wait()
            for m in range(_MA):
                pltpu.make_async_copy(
                    h_half.at[sidx[s][m]],
                    rows[b].at[pl.ds(m * _C, _C)], semg[b]).wait()

            def edge_body(i, c2):
                for j in range(_HALF // _LANES):
                    sl = pl.ds(j * _LANES, _LANES)
                    ep[b][i, sl] = jnp.maximum(rows[b][i, sl] + ep[b][i, sl],
                                               0.0)
                return c2

            lax.fori_loop(0, _SUPA, edge_body, 0)
            for m in range(_MA):
                pltpu.async_copy(ep[b].at[pl.ds(m * _C, _C)],
                                 acc_sh.at[didx[s][m]], sems[b], add=True)

        def drain_scatter(b, s):
            for m in range(_MA):
                pltpu.make_async_copy(ep[b].at[pl.ds(m * _C, _C)],
                                      acc_sh.at[didx[s][m]], sems[b]).wait()

        prefetch_idx(0, 0, 0)
        prefetch_idx(1, 1, 1)
        issue(0, 0, 0)
        issue(1, 1, 1)

        def body(kk, carry):
            for ph in range(4):
                b = ph % 2
                s2 = (ph + 2) % 4

                @pl.when((kk % 4) == ph)
                def _(b=b, s=ph, s2=s2):
                    @pl.when(kk + 2 < _NSUPA)
                    def _():
                        prefetch_idx(kk + 2, b, s2)
                    finish(kk, b, s)

                    @pl.when(kk + 2 < _NSUPA)
                    def _():
                        drain_scatter(b, s)
                        issue(kk + 2, b, s2)

            return carry

        lax.fori_loop(0, _NSUPA, body, 0)
        drain_scatter(0, (_NSUPA - 1) % 4 if (_NSUPA - 1) % 2 == 0
                      else (_NSUPA - 2) % 4)
        drain_scatter(1, (_NSUPA - 1) % 4 if (_NSUPA - 1) % 2 == 1
                      else (_NSUPA - 2) % 4)

    @pl.when(ci == 0)
    def _():
        run(h_lo, ep_lo_h)

    @pl.when(ci == 1)
    def _():
        run(h_hi, ep_hi_h)

    plsc.subcore_barrier()

    @pl.when(jnp.logical_and(si == 0, ci == 0))
    def _():
        pltpu.sync_copy(acc_sh, out_lo)

    @pl.when(jnp.logical_and(si == 0, ci == 1))
    def _():
        pltpu.sync_copy(acc_sh, out_hi)


def _sc_aggr(h_lo, h_hi, src, dst, ep_lo, ep_hi, zeros):
    kern = pl.kernel(
        _sc_aggr_body,
        out_type=[jax.ShapeDtypeStruct((_N, _HALF), _F32),
                  jax.ShapeDtypeStruct((_N, _HALF), _F32)],
        mesh=_sc_mesh(),
        scratch_types=(
            [pltpu.VMEM((_SUPA, _HALF), _F32)] * 4
            + [pltpu.VMEM((_C,), jnp.int32)] * (8 * _MA)
            + [pltpu.VMEM_SHARED((_N, _HALF), _F32)]
            + [pltpu.SemaphoreType.DMA] * 10
        ),
    )
    return kern(h_lo, h_hi, src, dst, ep_lo, ep_hi, zeros)


def _sc_head_body(p_lo, p_hi, q_lo, q_hi, src_h, dst_h,
                  P_lo, P_hi, Q_lo, Q_hi,
                  sidx_all, didx_all, prow0, prow1, qrow0, qrow1,
                  semp0, semp1, semq0, semq1, semw0, semw1):
    ci = lax.axis_index("c")
    si = lax.axis_index("s")
    base = si * _EPT

    pltpu.sync_copy(src_h.at[pl.ds(base, _EPT)], sidx_all)
    pltpu.sync_copy(dst_h.at[pl.ds(base, _EPT)], didx_all)

    def run(p_half, q_half, P_half, Q_half):
        prows = (prow0, prow1)
        qrows = (qrow0, qrow1)
        semps = (semp0, semp1)
        semqs = (semq0, semq1)
        semws = (semw0, semw1)

        def issue(kk, b):
            loc = kk * _SUP
            for m in range(_M):
                pltpu.async_copy(
                    p_half.at[sidx_all.at[pl.ds(loc + m * _C, _C)]],
                    prows[b].at[pl.ds(m * _C, _C)], semps[b])
                pltpu.async_copy(
                    q_half.at[didx_all.at[pl.ds(loc + m * _C, _C)]],
                    qrows[b].at[pl.ds(m * _C, _C)], semqs[b])

        def finish(kk, b):
            off = base + kk * _SUP
            loc = kk * _SUP
            for m in range(_M):
                pltpu.make_async_copy(
                    p_half.at[sidx_all.at[pl.ds(loc + m * _C, _C)]],
                    prows[b].at[pl.ds(m * _C, _C)], semps[b]).wait()
                pltpu.make_async_copy(
                    q_half.at[didx_all.at[pl.ds(loc + m * _C, _C)]],
                    qrows[b].at[pl.ds(m * _C, _C)], semqs[b]).wait()
            pltpu.async_copy(prows[b], P_half.at[pl.ds(off, _SUP)], semws[b])
            pltpu.async_copy(qrows[b], Q_half.at[pl.ds(off, _SUP)], semws[b])

        def drain_writes(kk, b):
            off = base + kk * _SUP
            pltpu.make_async_copy(
                prows[b], P_half.at[pl.ds(off, _SUP)], semws[b]).wait()
            pltpu.make_async_copy(
                qrows[b], Q_half.at[pl.ds(off, _SUP)], semws[b]).wait()

        issue(0, 0)
        issue(1, 1)

        def body(kk, carry):
            even = (kk % 2) == 0

            @pl.when(even)
            def _():
                finish(kk, 0)

                @pl.when(kk + 2 < _NSUP)
                def _():
                    drain_writes(kk, 0)
                    issue(kk + 2, 0)

            @pl.when(jnp.logical_not(even))
            def _():
                finish(kk, 1)

                @pl.when(kk + 2 < _NSUP)
                def _():
                    drain_writes(kk, 1)
                    issue(kk + 2, 1)

            return carry

        lax.fori_loop(0, _NSUP, body, 0)
        drain_writes(_NSUP - 2, 0)
        drain_writes(_NSUP - 1, 1)

    @pl.when(ci == 0)
    def _():
        run(p_lo, q_lo, P_lo, Q_lo)

    @pl.when(ci == 1)
    def _():
        run(p_hi, q_hi, P_hi, Q_hi)


def _sc_head_edges(p_lo, p_hi, q_lo, q_hi, src, dst):
    kern = pl.kernel(
        _sc_head_body,
        out_type=[jax.ShapeDtypeStruct((_E, _HALF), _F32)] * 4,
        mesh=_sc_mesh(),
        scratch_types=[
            pltpu.VMEM((_EPT,), jnp.int32),
            pltpu.VMEM((_EPT,), jnp.int32),
            pltpu.VMEM((_SUP, _HALF), _F32),
            pltpu.VMEM((_SUP, _HALF), _F32),
            pltpu.VMEM((_SUP, _HALF), _F32),
            pltpu.VMEM((_SUP, _HALF), _F32),
            pltpu.SemaphoreType.DMA,
            pltpu.SemaphoreType.DMA,
            pltpu.SemaphoreType.DMA,
            pltpu.SemaphoreType.DMA,
            pltpu.SemaphoreType.DMA,
            pltpu.SemaphoreType.DMA,
        ],
    )
    return kern(p_lo, p_hi, q_lo, q_hi, src, dst)


_BN = 1000
_BE = 2000


def _init_body(x_ref, w_ref, b_ref, lo_ref, hi_ref):
    acc = jnp.dot(x_ref[...], w_ref[...], preferred_element_type=_F32)
    acc = acc + b_ref[...]
    lo_ref[...] = acc[:, :_HALF]
    hi_ref[...] = acc[:, _HALF:]


def _tc_init(x, w, b):
    return pl.pallas_call(
        _init_body,
        grid=(_N // _BN,),
        in_specs=[
            pl.BlockSpec((_BN, _H), lambda i: (i, 0)),
            pl.BlockSpec((_H, _H), lambda i: (0, 0)),
            pl.BlockSpec((1, _H), lambda i: (0, 0)),
        ],
        out_specs=[
            pl.BlockSpec((_BN, _HALF), lambda i: (i, 0)),
            pl.BlockSpec((_BN, _HALF), lambda i: (i, 0)),
        ],
        out_shape=[jax.ShapeDtypeStruct((_N, _HALF), _F32),
                   jax.ShapeDtypeStruct((_N, _HALF), _F32)],
    )(x, w, b)


def _eproj_body(ea_ref, ew_ref, eb_ref, lw0_ref, lb0_ref, lw1_ref, lb1_ref,
                lw2_ref, lb2_ref, *out_refs):
    e = jnp.dot(ea_ref[...], ew_ref[...],
                preferred_element_type=_F32) + eb_ref[...]
    for l, (lw, lb) in enumerate(((lw0_ref, lb0_ref), (lw1_ref, lb1_ref),
                                  (lw2_ref, lb2_ref))):
        ep = jnp.dot(e, lw[...], preferred_element_type=_F32) + lb[...]
        out_refs[2 * l][...] = ep[:, :_HALF]
        out_refs[2 * l + 1][...] = ep[:, _HALF:]


def _tc_eproj(ea, ew, eb, lws, lbs):
    return pl.pallas_call(
        _eproj_body,
        grid=(_E // _BE,),
        in_specs=[
            pl.BlockSpec((_BE, 2), lambda i: (i, 0)),
            pl.BlockSpec((2, _H), lambda i: (0, 0)),
            pl.BlockSpec((1, _H), lambda i: (0, 0)),
            pl.BlockSpec((_H, _H), lambda i: (0, 0)),
            pl.BlockSpec((1, _H), lambda i: (0, 0)),
            pl.BlockSpec((_H, _H), lambda i: (0, 0)),
            pl.BlockSpec((1, _H), lambda i: (0, 0)),
            pl.BlockSpec((_H, _H), lambda i: (0, 0)),
            pl.BlockSpec((1, _H), lambda i: (0, 0)),
        ],
        out_specs=[pl.BlockSpec((_BE, _HALF), lambda i: (i, 0))] * 6,
        out_shape=[jax.ShapeDtypeStruct((_E, _HALF), _F32)] * 6,
    )(ea, ew, eb, lws[0], lbs[0], lws[1], lbs[1], lws[2], lbs[2])


def _mlp_body(hlo_ref, hhi_ref, alo_ref, ahi_ref, m1_ref, b1_ref,
              m2_ref, b2_ref, z2_ref, stats_ref):
    z = jnp.concatenate(
        [hlo_ref[...] + alo_ref[...], hhi_ref[...] + ahi_ref[...]], axis=1)
    t = jnp.dot(z, m1_ref[...], preferred_element_type=_F32) + b1_ref[...]
    t = jnp.maximum(t, 0.0)
    zz = jnp.dot(t, m2_ref[...], preferred_element_type=_F32) + b2_ref[...]
    z2_ref[...] = zz
    s = jnp.sum(zz, axis=0, keepdims=True)
    sq = jnp.sum(zz * zz, axis=0, keepdims=True)
    st = jnp.concatenate([s, sq], axis=0)

    @pl.when(pl.program_id(0) == 0)
    def _():
        stats_ref[...] = st

    @pl.when(pl.program_id(0) != 0)
    def _():
        stats_ref[...] = stats_ref[...] + st


def _tc_mlp(h_lo, h_hi, a_lo, a_hi, m1, b1, m2, b2):
    return pl.pallas_call(
        _mlp_body,
        grid=(_N // _BN,),
        in_specs=[
            pl.BlockSpec((_BN, _HALF), lambda i: (i, 0)),
            pl.BlockSpec((_BN, _HALF), lambda i: (i, 0)),
            pl.BlockSpec((_BN, _HALF), lambda i: (i, 0)),
            pl.BlockSpec((_BN, _HALF), lambda i: (i, 0)),
            pl.BlockSpec((_H, _H), lambda i: (0, 0)),
            pl.BlockSpec((1, _H), lambda i: (0, 0)),
            pl.BlockSpec((_H, _H), lambda i: (0, 0)),
            pl.BlockSpec((1, _H), lambda i: (0, 0)),
        ],
        out_specs=[
            pl.BlockSpec((_BN, _H), lambda i: (i, 0)),
            pl.BlockSpec((2, _H), lambda i: (0, 0)),
        ],
        out_shape=[jax.ShapeDtypeStruct((_N, _H), _F32),
                   jax.ShapeDtypeStruct((2, _H), _F32)],
    )(h_lo, h_hi, a_lo, a_hi, m1, b1, m2, b2)


def _bn_body(z2_ref, hlo_ref, hhi_ref, mu_ref, sc_ref, be_ref,
             lo_ref, hi_ref):
    h = jnp.concatenate([hlo_ref[...], hhi_ref[...]], axis=1)
    zn = (z2_ref[...] - mu_ref[...]) * sc_ref[...] + be_ref[...]
    nh = jnp.maximum(zn + h, 0.0)
    lo_ref[...] = nh[:, :_HALF]
    hi_ref[...] = nh[:, _HALF:]


def _tc_bn_residual(z2, h_lo, h_hi, mu, scale, beta):
    return pl.pallas_call(
        _bn_body,
        grid=(_N // _BN,),
        in_specs=[
            pl.BlockSpec((_BN, _H), lambda i: (i, 0)),
            pl.BlockSpec((_BN, _HALF), lambda i: (i, 0)),
            pl.BlockSpec((_BN, _HALF), lambda i: (i, 0)),
            pl.BlockSpec((1, _H), lambda i: (0, 0)),
            pl.BlockSpec((1, _H), lambda i: (0, 0)),
            pl.BlockSpec((1, _H), lambda i: (0, 0)),
        ],
        out_specs=[
            pl.BlockSpec((_BN, _HALF), lambda i: (i, 0)),
            pl.BlockSpec((_BN, _HALF), lambda i: (i, 0)),
        ],
        out_shape=[jax.ShapeDtypeStruct((_N, _HALF), _F32),
                   jax.ShapeDtypeStruct((_N, _HALF), _F32)],
    )(z2, h_lo, h_hi, mu, scale, beta)


def _proj_body(hlo_ref, hhi_ref, wa_ref, ba_ref, wb_ref,
               plo_ref, phi_ref, qlo_ref, qhi_ref):
    h = jnp.concatenate([hlo_ref[...], hhi_ref[...]], axis=1)
    p = jnp.dot(h, wa_ref[...], preferred_element_type=_F32) + ba_ref[...]
    q = jnp.dot(h, wb_ref[...], preferred_element_type=_F32)
    plo_ref[...] = p[:, :_HALF]
    phi_ref[...] = p[:, _HALF:]
    qlo_ref[...] = q[:, :_HALF]
    qhi_ref[...] = q[:, _HALF:]


def _tc_head_proj(h_lo, h_hi, wa, ba, wb):
    return pl.pallas_call(
        _proj_body,
        grid=(_N // _BN,),
        in_specs=[
            pl.BlockSpec((_BN, _HALF), lambda i: (i, 0)),
            pl.BlockSpec((_BN, _HALF), lambda i: (i, 0)),
            pl.BlockSpec((_H, _H), lambda i: (0, 0)),
            pl.BlockSpec((1, _H), lambda i: (0, 0)),
            pl.BlockSpec((_H, _H), lambda i: (0, 0)),
        ],
        out_specs=[pl.BlockSpec((_BN, _HALF), lambda i: (i, 0))] * 4,
        out_shape=[jax.ShapeDtypeStruct((_N, _HALF), _F32)] * 4,
    )(h_lo, h_hi, wa, ba, wb)


def _tail_body(Plo_ref, Phi_ref, Qlo_ref, Qhi_ref, ea_ref, uvc_ref,
               w2_ref, b2_ref, w3_ref, b3_ref, out_ref):
    s = jnp.concatenate([Plo_ref[...] + Qlo_ref[...],
                         Phi_ref[...] + Qhi_ref[...]], axis=1)
    a0 = ea_ref[:, 0:1]
    a1 = ea_ref[:, 1:2]
    r = a0 * uvc_ref[0:1, :] + a1 * uvc_ref[1:2, :] + uvc_ref[2:3, :]
    g = jnp.maximum(s + r, 0.0)
    t = jnp.dot(g, w2_ref[...], preferred_element_type=_F32) + b2_ref[...]
    t = jnp.maximum(t, 0.0)
    out_ref[...] = jnp.dot(t, w3_ref[...], preferred_element_type=_F32) \
        + b3_ref[...]


def _tc_head_tail(P_lo, P_hi, Q_lo, Q_hi, ea, uvc, w2, b2, w3, b3):
    hh = _H // 2
    out = 8
    return pl.pallas_call(
        _tail_body,
        grid=(_E // _BE,),
        in_specs=[
            pl.BlockSpec((_BE, _HALF), lambda i: (i, 0)),
            pl.BlockSpec((_BE, _HALF), lambda i: (i, 0)),
            pl.BlockSpec((_BE, _HALF), lambda i: (i, 0)),
            pl.BlockSpec((_BE, _HALF), lambda i: (i, 0)),
            pl.BlockSpec((_BE, 2), lambda i: (i, 0)),
            pl.BlockSpec((3, _H), lambda i: (0, 0)),
            pl.BlockSpec((_H, hh), lambda i: (0, 0)),
            pl.BlockSpec((1, hh), lambda i: (0, 0)),
            pl.BlockSpec((hh, out), lambda i: (0, 0)),
            pl.BlockSpec((1, out), lambda i: (0, 0)),
        ],
        out_specs=pl.BlockSpec((_BE, out), lambda i: (i, 0)),
        out_shape=jax.ShapeDtypeStruct((_E, out), _F32),
    )(P_lo, P_hi, Q_lo, Q_hi, ea, uvc, w2, b2, w3, b3)


def kernel(x, edge_index, edge_attr, params):
    src = edge_index[0]
    dst = edge_index[1]
    zeros = jnp.zeros((_N, _HALF), _F32)

    h_lo, h_hi = _tc_init(x, params["node_W"], params["node_b"][None, :])

    eps = _tc_eproj(
        edge_attr, params["edge_W"], params["edge_b"][None, :],
        [lp["lin_W"] for lp in params["layers"]],
        [lp["lin_b"][None, :] for lp in params["layers"]])

    for li, lp in enumerate(params["layers"]):
        a_lo, a_hi = _sc_aggr(h_lo, h_hi, src, dst,
                              eps[2 * li], eps[2 * li + 1], zeros)

        z2, stats = _tc_mlp(h_lo, h_hi, a_lo, a_hi,
                            lp["m1_W"], lp["m1_b"][None, :],
                            lp["m2_W"], lp["m2_b"][None, :])
        mu = stats[0] / _N
        var = stats[1] / _N - mu * mu
        scale = lp["bn_g"] / jnp.sqrt(var + 1e-5)
        h_lo, h_hi = _tc_bn_residual(z2, h_lo, h_hi, mu[None, :],
                                     scale[None, :], lp["bn_b"][None, :])

    wa = params["h1_W"][:_H]
    wb = params["h1_W"][_H:2 * _H]
    uvc_head = jnp.concatenate(
        [params["h1_W"][2 * _H:], params["h1_b"][None, :]], axis=0)

    p_lo, p_hi, q_lo, q_hi = _tc_head_proj(
        h_lo, h_hi, wa, params["h1_b"][None, :] * 0.0, wb)

    P_lo, P_hi, Q_lo, Q_hi = _sc_head_edges(
        p_lo, p_hi, q_lo, q_hi, src, dst)

    return _tc_head_tail(P_lo, P_hi, Q_lo, Q_hi, edge_attr, uvc_head,
                         params["h2_W"], params["h2_b"][None, :],
                         params["h3_W"], params["h3_b"][None, :])


# speedup vs baseline: 3.3806x; 1.0280x over previous
import functools

import jax
import jax.numpy as jnp
from jax import lax
from jax.experimental import pallas as pl
from jax.experimental.pallas import tpu as pltpu
from jax.experimental.pallas import tpu_sc as plsc

_N = 10000
_E = 160000
_H = 256
_HALF = 128
_NC = 2
_NS = 16
_C = 40
_EPT = _E // _NS
_LANES = 16
_F32 = jnp.float32


def _sc_mesh():
    return plsc.VectorSubcoreMesh(
        core_axis_name="c", subcore_axis_name="s",
        num_cores=_NC, num_subcores=_NS)


_M = 5
_SUP = _C * _M
_NSUP = _EPT // _SUP
_CPS = _EPT // _C


_MA = 2
_SUPA = _C * _MA
_NSUPA = _EPT // _SUPA


def _sc_aggr_body(h_lo, h_hi, src_h, dst_h, ep_lo_h, ep_hi_h,
                  zeros_h, out_lo, out_hi, *sc):
    buf = sc[0:4]
    sidx = tuple(sc[4 + s * _MA:4 + (s + 1) * _MA] for s in range(4))
    didx = tuple(sc[12 + s * _MA:12 + (s + 1) * _MA] for s in range(4))
    acc_sh = sc[20]
    seme = sc[21:25]
    semg = sc[25:29]
    sems = sc[29:33]
    semi = sc[33:37]

    ci = lax.axis_index("c")
    si = lax.axis_index("s")
    base = si * _EPT

    @pl.when(si == 0)
    def _():
        pltpu.sync_copy(zeros_h, acc_sh)
    plsc.subcore_barrier()

    def run(h_half, ep_half):
        def prefetch_idx(kk, s):
            off = base + kk * _SUPA
            for m in range(_MA):
                pltpu.async_copy(src_h.at[pl.ds(off + m * _C, _C)],
                                 sidx[s][m], semi[s])
                pltpu.async_copy(dst_h.at[pl.ds(off + m * _C, _C)],
                                 didx[s][m], semi[s])

        def ep_copy(kk, b):
            off = base + kk * _SUPA
            pltpu.async_copy(ep_half.at[pl.ds(off, _SUPA)], buf[b], seme[b])

        def gather_add(kk, b):
            off = base + kk * _SUPA
            pltpu.make_async_copy(
                ep_half.at[pl.ds(off, _SUPA)], buf[b], seme[b]).wait()
            for m in range(_MA):
                pltpu.make_async_copy(src_h.at[pl.ds(off + m * _C, _C)],
                                      sidx[b][m], semi[b]).wait()
                pltpu.make_async_copy(dst_h.at[pl.ds(off + m * _C, _C)],
                                      didx[b][m], semi[b]).wait()
                pltpu.async_copy(h_half.at[sidx[b][m]],
                                 buf[b].at[pl.ds(m * _C, _C)], semg[b],
                                 add=True)

        def finish(kk, b):
            for m in range(_MA):
                pltpu.make_async_copy(h_half.at[sidx[b][m]],
                                      buf[b].at[pl.ds(m * _C, _C)],
                                      semg[b]).wait()

            def edge_body(i, c2):
                for j in range(_HALF // _LANES):
                    sl = pl.ds(j * _LANES, _LANES)
                    buf[b][i, sl] = jnp.maximum(buf[b][i, sl], 0.0)
                return c2

            lax.fori_loop(0, _SUPA, edge_body, 0)
            for m in range(_MA):
                pltpu.async_copy(buf[b].at[pl.ds(m * _C, _C)],
                                 acc_sh.at[didx[b][m]], sems[b], add=True)

        def drain_scatter(b):
            for m in range(_MA):
                pltpu.make_async_copy(buf[b].at[pl.ds(m * _C, _C)],
                                      acc_sh.at[didx[b][m]], sems[b]).wait()

        prefetch_idx(0, 0)
        prefetch_idx(1, 1)
        ep_copy(0, 0)
        ep_copy(1, 1)
        gather_add(0, 0)

        def body(kk, carry):
            for ph in range(4):
                b2 = (ph + 2) % 4
                b1 = (ph + 1) % 4

                @pl.when((kk % 4) == ph)
                def _(b=ph, b1=b1, b2=b2):
                    @pl.when(kk - 2 >= 0)
                    def _():
                        drain_scatter(b2)

                    @pl.when(kk + 2 < _NSUPA)
                    def _():
                        prefetch_idx(kk + 2, b2)
                        ep_copy(kk + 2, b2)

                    @pl.when(kk + 1 < _NSUPA)
                    def _():
                        gather_add(kk + 1, b1)
                    finish(kk, b)

            return carry

        lax.fori_loop(0, _NSUPA, body, 0)
        drain_scatter((_NSUPA - 2) % 4)
        drain_scatter((_NSUPA - 1) % 4)

    @pl.when(ci == 0)
    def _():
        run(h_lo, ep_lo_h)

    @pl.when(ci == 1)
    def _():
        run(h_hi, ep_hi_h)

    plsc.subcore_barrier()

    @pl.when(jnp.logical_and(si == 0, ci == 0))
    def _():
        pltpu.sync_copy(acc_sh, out_lo)

    @pl.when(jnp.logical_and(si == 0, ci == 1))
    def _():
        pltpu.sync_copy(acc_sh, out_hi)


def _sc_aggr(h_lo, h_hi, src, dst, ep_lo, ep_hi, zeros):
    kern = pl.kernel(
        _sc_aggr_body,
        out_type=[jax.ShapeDtypeStruct((_N, _HALF), _F32),
                  jax.ShapeDtypeStruct((_N, _HALF), _F32)],
        mesh=_sc_mesh(),
        scratch_types=(
            [pltpu.VMEM((_SUPA, _HALF), _F32)] * 4
            + [pltpu.VMEM((_C,), jnp.int32)] * (8 * _MA)
            + [pltpu.VMEM_SHARED((_N, _HALF), _F32)]
            + [pltpu.SemaphoreType.DMA] * 16
        ),
    )
    return kern(h_lo, h_hi, src, dst, ep_lo, ep_hi, zeros)


def _sc_head_body(p_lo, p_hi, q_lo, q_hi, src_h, dst_h,
                  P_lo, P_hi, Q_lo, Q_hi,
                  sidx_all, didx_all, prow0, prow1, qrow0, qrow1,
                  semp0, semp1, semq0, semq1, semw0, semw1):
    ci = lax.axis_index("c")
    si = lax.axis_index("s")
    base = si * _EPT

    pltpu.sync_copy(src_h.at[pl.ds(base, _EPT)], sidx_all)
    pltpu.sync_copy(dst_h.at[pl.ds(base, _EPT)], didx_all)

    def run(p_half, q_half, P_half, Q_half):
        prows = (prow0, prow1)
        qrows = (qrow0, qrow1)
        semps = (semp0, semp1)
        semqs = (semq0, semq1)
        semws = (semw0, semw1)

        def issue(kk, b):
            loc = kk * _SUP
            for m in range(_M):
                pltpu.async_copy(
                    p_half.at[sidx_all.at[pl.ds(loc + m * _C, _C)]],
                    prows[b].at[pl.ds(m * _C, _C)], semps[b])
                pltpu.async_copy(
                    q_half.at[didx_all.at[pl.ds(loc + m * _C, _C)]],
                    qrows[b].at[pl.ds(m * _C, _C)], semqs[b])

        def finish(kk, b):
            off = base + kk * _SUP
            loc = kk * _SUP
            for m in range(_M):
                pltpu.make_async_copy(
                    p_half.at[sidx_all.at[pl.ds(loc + m * _C, _C)]],
                    prows[b].at[pl.ds(m * _C, _C)], semps[b]).wait()
                pltpu.make_async_copy(
                    q_half.at[didx_all.at[pl.ds(loc + m * _C, _C)]],
                    qrows[b].at[pl.ds(m * _C, _C)], semqs[b]).wait()
            pltpu.async_copy(prows[b], P_half.at[pl.ds(off, _SUP)], semws[b])
            pltpu.async_copy(qrows[b], Q_half.at[pl.ds(off, _SUP)], semws[b])

        def drain_writes(kk, b):
            off = base + kk * _SUP
            pltpu.make_async_copy(
                prows[b], P_half.at[pl.ds(off, _SUP)], semws[b]).wait()
            pltpu.make_async_copy(
                qrows[b], Q_half.at[pl.ds(off, _SUP)], semws[b]).wait()

        issue(0, 0)
        issue(1, 1)

        def body(kk, carry):
            even = (kk % 2) == 0

            @pl.when(even)
            def _():
                finish(kk, 0)

                @pl.when(kk + 2 < _NSUP)
                def _():
                    drain_writes(kk, 0)
                    issue(kk + 2, 0)

            @pl.when(jnp.logical_not(even))
            def _():
                finish(kk, 1)

                @pl.when(kk + 2 < _NSUP)
                def _():
                    drain_writes(kk, 1)
                    issue(kk + 2, 1)

            return carry

        lax.fori_loop(0, _NSUP, body, 0)
        drain_writes(_NSUP - 2, 0)
        drain_writes(_NSUP - 1, 1)

    @pl.when(ci == 0)
    def _():
        run(p_lo, q_lo, P_lo, Q_lo)

    @pl.when(ci == 1)
    def _():
        run(p_hi, q_hi, P_hi, Q_hi)


def _sc_head_edges(p_lo, p_hi, q_lo, q_hi, src, dst):
    kern = pl.kernel(
        _sc_head_body,
        out_type=[jax.ShapeDtypeStruct((_E, _HALF), _F32)] * 4,
        mesh=_sc_mesh(),
        scratch_types=[
            pltpu.VMEM((_EPT,), jnp.int32),
            pltpu.VMEM((_EPT,), jnp.int32),
            pltpu.VMEM((_SUP, _HALF), _F32),
            pltpu.VMEM((_SUP, _HALF), _F32),
            pltpu.VMEM((_SUP, _HALF), _F32),
            pltpu.VMEM((_SUP, _HALF), _F32),
            pltpu.SemaphoreType.DMA,
            pltpu.SemaphoreType.DMA,
            pltpu.SemaphoreType.DMA,
            pltpu.SemaphoreType.DMA,
            pltpu.SemaphoreType.DMA,
            pltpu.SemaphoreType.DMA,
        ],
    )
    return kern(p_lo, p_hi, q_lo, q_hi, src, dst)


_BN = 1000
_BE = 2000


def _init_body(x_ref, w_ref, b_ref, lo_ref, hi_ref):
    acc = jnp.dot(x_ref[...], w_ref[...], preferred_element_type=_F32)
    acc = acc + b_ref[...]
    lo_ref[...] = acc[:, :_HALF]
    hi_ref[...] = acc[:, _HALF:]


def _tc_init(x, w, b):
    return pl.pallas_call(
        _init_body,
        grid=(_N // _BN,),
        in_specs=[
            pl.BlockSpec((_BN, _H), lambda i: (i, 0)),
            pl.BlockSpec((_H, _H), lambda i: (0, 0)),
            pl.BlockSpec((1, _H), lambda i: (0, 0)),
        ],
        out_specs=[
            pl.BlockSpec((_BN, _HALF), lambda i: (i, 0)),
            pl.BlockSpec((_BN, _HALF), lambda i: (i, 0)),
        ],
        out_shape=[jax.ShapeDtypeStruct((_N, _HALF), _F32),
                   jax.ShapeDtypeStruct((_N, _HALF), _F32)],
    )(x, w, b)


def _eproj_body(ea_ref, ew_ref, eb_ref, lw0_ref, lb0_ref, lw1_ref, lb1_ref,
                lw2_ref, lb2_ref, *out_refs):
    e = jnp.dot(ea_ref[...], ew_ref[...],
                preferred_element_type=_F32) + eb_ref[...]
    for l, (lw, lb) in enumerate(((lw0_ref, lb0_ref), (lw1_ref, lb1_ref),
                                  (lw2_ref, lb2_ref))):
        ep = jnp.dot(e, lw[...], preferred_element_type=_F32) + lb[...]
        out_refs[2 * l][...] = ep[:, :_HALF]
        out_refs[2 * l + 1][...] = ep[:, _HALF:]


def _tc_eproj(ea, ew, eb, lws, lbs):
    return pl.pallas_call(
        _eproj_body,
        grid=(_E // _BE,),
        in_specs=[
            pl.BlockSpec((_BE, 2), lambda i: (i, 0)),
            pl.BlockSpec((2, _H), lambda i: (0, 0)),
            pl.BlockSpec((1, _H), lambda i: (0, 0)),
            pl.BlockSpec((_H, _H), lambda i: (0, 0)),
            pl.BlockSpec((1, _H), lambda i: (0, 0)),
            pl.BlockSpec((_H, _H), lambda i: (0, 0)),
            pl.BlockSpec((1, _H), lambda i: (0, 0)),
            pl.BlockSpec((_H, _H), lambda i: (0, 0)),
            pl.BlockSpec((1, _H), lambda i: (0, 0)),
        ],
        out_specs=[pl.BlockSpec((_BE, _HALF), lambda i: (i, 0))] * 6,
        out_shape=[jax.ShapeDtypeStruct((_E, _HALF), _F32)] * 6,
    )(ea, ew, eb, lws[0], lbs[0], lws[1], lbs[1], lws[2], lbs[2])


def _mlp_body(hlo_ref, hhi_ref, alo_ref, ahi_ref, m1_ref, b1_ref,
              m2_ref, b2_ref, z2_ref, stats_ref):
    z = jnp.concatenate(
        [hlo_ref[...] + alo_ref[...], hhi_ref[...] + ahi_ref[...]], axis=1)
    t = jnp.dot(z, m1_ref[...], preferred_element_type=_F32) + b1_ref[...]
    t = jnp.maximum(t, 0.0)
    zz = jnp.dot(t, m2_ref[...], preferred_element_type=_F32) + b2_ref[...]
    z2_ref[...] = zz
    s = jnp.sum(zz, axis=0, keepdims=True)
    sq = jnp.sum(zz * zz, axis=0, keepdims=True)
    st = jnp.concatenate([s, sq], axis=0)

    @pl.when(pl.program_id(0) == 0)
    def _():
        stats_ref[...] = st

    @pl.when(pl.program_id(0) != 0)
    def _():
        stats_ref[...] = stats_ref[...] + st


def _tc_mlp(h_lo, h_hi, a_lo, a_hi, m1, b1, m2, b2):
    return pl.pallas_call(
        _mlp_body,
        grid=(_N // _BN,),
        in_specs=[
            pl.BlockSpec((_BN, _HALF), lambda i: (i, 0)),
            pl.BlockSpec((_BN, _HALF), lambda i: (i, 0)),
            pl.BlockSpec((_BN, _HALF), lambda i: (i, 0)),
            pl.BlockSpec((_BN, _HALF), lambda i: (i, 0)),
            pl.BlockSpec((_H, _H), lambda i: (0, 0)),
            pl.BlockSpec((1, _H), lambda i: (0, 0)),
            pl.BlockSpec((_H, _H), lambda i: (0, 0)),
            pl.BlockSpec((1, _H), lambda i: (0, 0)),
        ],
        out_specs=[
            pl.BlockSpec((_BN, _H), lambda i: (i, 0)),
            pl.BlockSpec((2, _H), lambda i: (0, 0)),
        ],
        out_shape=[jax.ShapeDtypeStruct((_N, _H), _F32),
                   jax.ShapeDtypeStruct((2, _H), _F32)],
    )(h_lo, h_hi, a_lo, a_hi, m1, b1, m2, b2)


def _bn_body(z2_ref, hlo_ref, hhi_ref, mu_ref, sc_ref, be_ref,
             lo_ref, hi_ref):
    h = jnp.concatenate([hlo_ref[...], hhi_ref[...]], axis=1)
    zn = (z2_ref[...] - mu_ref[...]) * sc_ref[...] + be_ref[...]
    nh = jnp.maximum(zn + h, 0.0)
    lo_ref[...] = nh[:, :_HALF]
    hi_ref[...] = nh[:, _HALF:]


def _tc_bn_residual(z2, h_lo, h_hi, mu, scale, beta):
    return pl.pallas_call(
        _bn_body,
        grid=(_N // _BN,),
        in_specs=[
            pl.BlockSpec((_BN, _H), lambda i: (i, 0)),
            pl.BlockSpec((_BN, _HALF), lambda i: (i, 0)),
            pl.BlockSpec((_BN, _HALF), lambda i: (i, 0)),
            pl.BlockSpec((1, _H), lambda i: (0, 0)),
            pl.BlockSpec((1, _H), lambda i: (0, 0)),
            pl.BlockSpec((1, _H), lambda i: (0, 0)),
        ],
        out_specs=[
            pl.BlockSpec((_BN, _HALF), lambda i: (i, 0)),
            pl.BlockSpec((_BN, _HALF), lambda i: (i, 0)),
        ],
        out_shape=[jax.ShapeDtypeStruct((_N, _HALF), _F32),
                   jax.ShapeDtypeStruct((_N, _HALF), _F32)],
    )(z2, h_lo, h_hi, mu, scale, beta)


def _proj_body(hlo_ref, hhi_ref, wa_ref, ba_ref, wb_ref,
               plo_ref, phi_ref, qlo_ref, qhi_ref):
    h = jnp.concatenate([hlo_ref[...], hhi_ref[...]], axis=1)
    p = jnp.dot(h, wa_ref[...], preferred_element_type=_F32) + ba_ref[...]
    q = jnp.dot(h, wb_ref[...], preferred_element_type=_F32)
    plo_ref[...] = p[:, :_HALF]
    phi_ref[...] = p[:, _HALF:]
    qlo_ref[...] = q[:, :_HALF]
    qhi_ref[...] = q[:, _HALF:]


def _tc_head_proj(h_lo, h_hi, wa, ba, wb):
    return pl.pallas_call(
        _proj_body,
        grid=(_N // _BN,),
        in_specs=[
            pl.BlockSpec((_BN, _HALF), lambda i: (i, 0)),
            pl.BlockSpec((_BN, _HALF), lambda i: (i, 0)),
            pl.BlockSpec((_H, _H), lambda i: (0, 0)),
            pl.BlockSpec((1, _H), lambda i: (0, 0)),
            pl.BlockSpec((_H, _H), lambda i: (0, 0)),
        ],
        out_specs=[pl.BlockSpec((_BN, _HALF), lambda i: (i, 0))] * 4,
        out_shape=[jax.ShapeDtypeStruct((_N, _HALF), _F32)] * 4,
    )(h_lo, h_hi, wa, ba, wb)


def _tail_body(Plo_ref, Phi_ref, Qlo_ref, Qhi_ref, ea_ref, uvc_ref,
               w2_ref, b2_ref, w3_ref, b3_ref, out_ref):
    s = jnp.concatenate([Plo_ref[...] + Qlo_ref[...],
                         Phi_ref[...] + Qhi_ref[...]], axis=1)
    a0 = ea_ref[:, 0:1]
    a1 = ea_ref[:, 1:2]
    r = a0 * uvc_ref[0:1, :] + a1 * uvc_ref[1:2, :] + uvc_ref[2:3, :]
    g = jnp.maximum(s + r, 0.0)
    t = jnp.dot(g, w2_ref[...], preferred_element_type=_F32) + b2_ref[...]
    t = jnp.maximum(t, 0.0)
    out_ref[...] = jnp.dot(t, w3_ref[...], preferred_element_type=_F32) \
        + b3_ref[...]


def _tc_head_tail(P_lo, P_hi, Q_lo, Q_hi, ea, uvc, w2, b2, w3, b3):
    hh = _H // 2
    out = 8
    return pl.pallas_call(
        _tail_body,
        grid=(_E // _BE,),
        in_specs=[
            pl.BlockSpec((_BE, _HALF), lambda i: (i, 0)),
            pl.BlockSpec((_BE, _HALF), lambda i: (i, 0)),
            pl.BlockSpec((_BE, _HALF), lambda i: (i, 0)),
            pl.BlockSpec((_BE, _HALF), lambda i: (i, 0)),
            pl.BlockSpec((_BE, 2), lambda i: (i, 0)),
            pl.BlockSpec((3, _H), lambda i: (0, 0)),
            pl.BlockSpec((_H, hh), lambda i: (0, 0)),
            pl.BlockSpec((1, hh), lambda i: (0, 0)),
            pl.BlockSpec((hh, out), lambda i: (0, 0)),
            pl.BlockSpec((1, out), lambda i: (0, 0)),
        ],
        out_specs=pl.BlockSpec((_BE, out), lambda i: (i, 0)),
        out_shape=jax.ShapeDtypeStruct((_E, out), _F32),
    )(P_lo, P_hi, Q_lo, Q_hi, ea, uvc, w2, b2, w3, b3)


def kernel(x, edge_index, edge_attr, params):
    src = edge_index[0]
    dst = edge_index[1]
    zeros = jnp.zeros((_N, _HALF), _F32)

    h_lo, h_hi = _tc_init(x, params["node_W"], params["node_b"][None, :])

    eps = _tc_eproj(
        edge_attr, params["edge_W"], params["edge_b"][None, :],
        [lp["lin_W"] for lp in params["layers"]],
        [lp["lin_b"][None, :] for lp in params["layers"]])

    for li, lp in enumerate(params["layers"]):
        a_lo, a_hi = _sc_aggr(h_lo, h_hi, src, dst,
                              eps[2 * li], eps[2 * li + 1], zeros)

        z2, stats = _tc_mlp(h_lo, h_hi, a_lo, a_hi,
                            lp["m1_W"], lp["m1_b"][None, :],
                            lp["m2_W"], lp["m2_b"][None, :])
        mu = stats[0] / _N
        var = stats[1] / _N - mu * mu
        scale = lp["bn_g"] / jnp.sqrt(var + 1e-5)
        h_lo, h_hi = _tc_bn_residual(z2, h_lo, h_hi, mu[None, :],
                                     scale[None, :], lp["bn_b"][None, :])

    wa = params["h1_W"][:_H]
    wb = params["h1_W"][_H:2 * _H]
    uvc_head = jnp.concatenate(
        [params["h1_W"][2 * _H:], params["h1_b"][None, :]], axis=0)

    p_lo, p_hi, q_lo, q_hi = _tc_head_proj(
        h_lo, h_hi, wa, params["h1_b"][None, :] * 0.0, wb)

    P_lo, P_hi, Q_lo, Q_hi = _sc_head_edges(
        p_lo, p_hi, q_lo, q_hi, src, dst)

    return _tc_head_tail(P_lo, P_hi, Q_lo, Q_hi, edge_attr, uvc_head,
                         params["h2_W"], params["h2_b"][None, :],
                         params["h3_W"], params["h3_b"][None, :])
